# baseline (speedup 1.0000x reference)
import functools

import jax
import jax.numpy as jnp
from jax import lax
from jax.experimental import pallas as pl
from jax.experimental.pallas import tpu as pltpu
from jax.experimental.pallas import tpu_sc as plsc

_B = 4096
_F = 26
_V = 100000
_D = 16
_ND = 13
_HID = 512
_BF = _B * _F
_NW = 32
_CHUNK = _BF // _NW
_NG = _CHUNK // 16
_BB = 256


def _gather_embs(idx2, tab1):
    mesh = plsc.VectorSubcoreMesh(core_axis_name="c", subcore_axis_name="s")

    @functools.partial(
        pl.kernel,
        out_type=jax.ShapeDtypeStruct((_BF * _D,), jnp.float32),
        mesh=mesh,
        compiler_params=pltpu.CompilerParams(use_tc_tiling_on_sc=False),
        scratch_types=[
            pltpu.VMEM((_CHUNK,), jnp.int32),
            pltpu.VMEM((_CHUNK * _D,), jnp.float32),
            pltpu.SemaphoreType.DMA,
        ],
    )
    def sc_gather(idx_hbm, tab_hbm, out_hbm, idx_v, rows_v, sem):
        wid = lax.axis_index("s") * 2 + lax.axis_index("c")
        base = wid * _CHUNK
        pltpu.sync_copy(idx_hbm.at[wid], idx_v)
        lane = lax.iota(jnp.int32, 16)

        def body(j, carry):
            v = idx_v[pl.ds(j * 16, 16)]
            f = lax.rem(base + j * 16 + lane, _F)
            wa = (v + f * _V) * _D
            for k in range(16):
                pltpu.async_copy(
                    tab_hbm.at[wa[k] + lane],
                    rows_v.at[pl.ds((j * 16 + k) * _D, _D)],
                    sem,
                )
            return carry

        lax.fori_loop(0, _NG, body, 0)
        pltpu.make_async_copy(
            out_hbm.at[pl.ds(base * _D, _CHUNK * _D)],
            rows_v,
            sem,
        ).wait()
        pltpu.sync_copy(rows_v, out_hbm.at[pl.ds(base * _D, _CHUNK * _D)])

    return sc_gather(idx2, tab1)


def _tc_body(x_ref, e_ref, w1_ref, b1_ref, w2_ref, b2_ref, wc_ref, wd_ref,
             bo_ref, o_ref):
    h = jnp.maximum(
        jnp.dot(x_ref[...], w1_ref[...], preferred_element_type=jnp.float32)
        + b1_ref[...], 0.0)
    dense = jnp.maximum(
        jnp.dot(h, w2_ref[...], preferred_element_type=jnp.float32)
        + b2_ref[...], 0.0)
    e = e_ref[...]
    gu = jnp.dot(e, wc_ref[...], preferred_element_type=jnp.float32)
    g = gu[:, : _F * _D]
    u = gu[:, _F * _D:]
    pd = jnp.sum(g * e, axis=1, keepdims=True)
    dterm = jnp.sum(dense * (u + wd_ref[...]), axis=1, keepdims=True)
    o_ref[...] = jnp.maximum(pd + dterm + bo_ref[...], 0.0)


def kernel(dense_features, sparse_indices, tables, W1, b1, W2, b2, Wo, bo):
    tab2 = lax.optimization_barrier(tables.reshape(_F * _V, _D))
    tab1 = tab2.reshape(_F * _V * _D)
    idx2 = sparse_indices.reshape(_NW, _CHUNK)
    embs = _gather_embs(idx2, tab1)
    e2 = embs.reshape(_B, _F * _D)

    wd = Wo[:_D, 0].reshape(1, _D)
    wsd = Wo[_D:_D + _F, 0]
    wpd = Wo[_D + _F:, 0]
    iu0, iu1 = jnp.triu_indices(_F, k=1)
    a = jnp.zeros((_F, _F), jnp.float32).at[iu0, iu1].set(wpd)
    a = 0.5 * (a + a.T)
    eye = jnp.eye(_D, dtype=jnp.float32)
    wc = jnp.concatenate(
        [jnp.kron(a, eye), jnp.kron(wsd[:, None], eye)], axis=1)

    out = pl.pallas_call(
        _tc_body,
        grid=(_B // _BB,),
        in_specs=[
            pl.BlockSpec((_BB, _ND), lambda i: (i, 0)),
            pl.BlockSpec((_BB, _F * _D), lambda i: (i, 0)),
            pl.BlockSpec((_ND, _HID), lambda i: (0, 0)),
            pl.BlockSpec((1, _HID), lambda i: (0, 0)),
            pl.BlockSpec((_HID, _D), lambda i: (0, 0)),
            pl.BlockSpec((1, _D), lambda i: (0, 0)),
            pl.BlockSpec((_F * _D, _F * _D + _D), lambda i: (0, 0)),
            pl.BlockSpec((1, _D), lambda i: (0, 0)),
            pl.BlockSpec((1, 1), lambda i: (0, 0)),
        ],
        out_specs=pl.BlockSpec((_BB, 1), lambda i: (i, 0)),
        out_shape=jax.ShapeDtypeStruct((_B, 1), jnp.float32),
    )(dense_features, e2, W1, b1.reshape(1, _HID), W2, b2.reshape(1, _D),
      wc, wd, bo.reshape(1, 1))
    return out


# speedup vs baseline: 6.1902x; 6.1902x over previous
import functools

import jax
import jax.numpy as jnp
from jax import lax
from jax.experimental import pallas as pl
from jax.experimental.pallas import tpu as pltpu
from jax.experimental.pallas import tpu_sc as plsc

_B = 4096
_F = 26
_V = 100000
_D = 16
_ND = 13
_HID = 512
_BF = _B * _F
_NW = 32
_CHUNK = _BF // _NW
_NG = _CHUNK // 16
_BB = 256


def _gather_embs(idx2, tab2):
    mesh = plsc.VectorSubcoreMesh(core_axis_name="c", subcore_axis_name="s")

    @functools.partial(
        pl.kernel,
        out_type=jax.ShapeDtypeStruct((_BF, _D), jnp.float32),
        mesh=mesh,
        compiler_params=pltpu.CompilerParams(use_tc_tiling_on_sc=False),
        scratch_types=[
            pltpu.VMEM((_CHUNK,), jnp.int32),
            pltpu.VMEM((_CHUNK, _D), jnp.float32),
            pltpu.SemaphoreType.DMA,
        ],
    )
    def sc_gather(idx_hbm, tab_hbm, out_hbm, idx_v, rows_v, sem):
        wid = lax.axis_index("s") * 2 + lax.axis_index("c")
        base = wid * _CHUNK
        pltpu.sync_copy(idx_hbm.at[wid], idx_v)
        lane = lax.iota(jnp.int32, 16)

        def body(j, carry):
            v = idx_v[pl.ds(j * 16, 16)]
            f = lax.rem(base + j * 16 + lane, _F)
            pltpu.async_copy(
                tab_hbm.at[v + f * _V],
                rows_v.at[pl.ds(j * 16, 16)],
                sem,
            )
            return carry

        lax.fori_loop(0, _NG, body, 0)
        pltpu.make_async_copy(
            out_hbm.at[pl.ds(base, _CHUNK)],
            rows_v,
            sem,
        ).wait()
        pltpu.sync_copy(rows_v, out_hbm.at[pl.ds(base, _CHUNK)])

    return sc_gather(idx2, tab2)


def _tc_body(x_ref, e_ref, w1_ref, b1_ref, w2_ref, b2_ref, wc_ref, wd_ref,
             bo_ref, o_ref):
    h = jnp.maximum(
        jnp.dot(x_ref[...], w1_ref[...], preferred_element_type=jnp.float32)
        + b1_ref[...], 0.0)
    dense = jnp.maximum(
        jnp.dot(h, w2_ref[...], preferred_element_type=jnp.float32)
        + b2_ref[...], 0.0)
    e = e_ref[...]
    gu = jnp.dot(e, wc_ref[...], preferred_element_type=jnp.float32)
    g = gu[:, : _F * _D]
    u = gu[:, _F * _D:]
    pd = jnp.sum(g * e, axis=1, keepdims=True)
    dterm = jnp.sum(dense * (u + wd_ref[...]), axis=1, keepdims=True)
    o_ref[...] = jnp.maximum(pd + dterm + bo_ref[...], 0.0)


def kernel(dense_features, sparse_indices, tables, W1, b1, W2, b2, Wo, bo):
    tab2 = tables.reshape(_F * _V, _D)
    idx2 = sparse_indices.reshape(_NW, _CHUNK)
    embs = _gather_embs(idx2, tab2)
    e2 = embs.reshape(_B, _F * _D)

    wd = Wo[:_D, 0].reshape(1, _D)
    wsd = Wo[_D:_D + _F, 0]
    wpd = Wo[_D + _F:, 0]
    iu0, iu1 = jnp.triu_indices(_F, k=1)
    a = jnp.zeros((_F, _F), jnp.float32).at[iu0, iu1].set(wpd)
    a = 0.5 * (a + a.T)
    eye = jnp.eye(_D, dtype=jnp.float32)
    wc = jnp.concatenate(
        [jnp.kron(a, eye), jnp.kron(wsd[:, None], eye)], axis=1)

    out = pl.pallas_call(
        _tc_body,
        grid=(_B // _BB,),
        in_specs=[
            pl.BlockSpec((_BB, _ND), lambda i: (i, 0)),
            pl.BlockSpec((_BB, _F * _D), lambda i: (i, 0)),
            pl.BlockSpec((_ND, _HID), lambda i: (0, 0)),
            pl.BlockSpec((1, _HID), lambda i: (0, 0)),
            pl.BlockSpec((_HID, _D), lambda i: (0, 0)),
            pl.BlockSpec((1, _D), lambda i: (0, 0)),
            pl.BlockSpec((_F * _D, _F * _D + _D), lambda i: (0, 0)),
            pl.BlockSpec((1, _D), lambda i: (0, 0)),
            pl.BlockSpec((1, 1), lambda i: (0, 0)),
        ],
        out_specs=pl.BlockSpec((_BB, 1), lambda i: (i, 0)),
        out_shape=jax.ShapeDtypeStruct((_B, 1), jnp.float32),
    )(dense_features, e2, W1, b1.reshape(1, _HID), W2, b2.reshape(1, _D),
      wc, wd, bo.reshape(1, 1))
    return out
